# baseline (speedup 1.0000x reference)
import jax
import jax.numpy as jnp
from jax.experimental import pallas as pl
from jax.experimental.pallas import tpu as pltpu

N = 50000
E = 800000
ATOM_EMB = 64
BOND_EMB = 16
MSG = 64
PASSES = 3
HID = 64
NGRAPH = 1024


def _mlp_head_body(mol_ref, w1_ref, b1_ref, w2_ref, b2_ref, out_ref):
    mol = mol_ref[...]
    h = jnp.maximum(
        jnp.dot(mol, w1_ref[...], preferred_element_type=jnp.float32)
        + b1_ref[...][None, :], 0.0)
    out = jnp.dot(h, w2_ref[...], preferred_element_type=jnp.float32) \
        + b2_ref[...][None, :]
    out_ref[...] = out[:, 0]


def _mlp_head(mol_state, W1, b1, W2, b2):
    return pl.pallas_call(
        _mlp_head_body,
        out_shape=jax.ShapeDtypeStruct((NGRAPH,), jnp.float32),
    )(mol_state, W1, b1, W2, b2)


def kernel(x, edge_index, edge_attr, batch, atom_table, bond_table,
           W_init, b_init, W_upd, b_upd, W1, b1, W2, b2):
    src = edge_index[0]
    dst = edge_index[1]

    A_cat = atom_table @ jnp.concatenate(
        [W_init[:ATOM_EMB], W_upd[:ATOM_EMB]], axis=1)
    nodeC = jnp.take(A_cat, x, axis=0)
    C = nodeC[:, :MSG]
    baseN = nodeC[:, MSG:]
    D_init = bond_table @ W_init[ATOM_EMB:] + b_init
    D_upd = bond_table @ W_upd[ATOM_EMB:ATOM_EMB + BOND_EMB] + b_upd
    Wa = W_upd[ATOM_EMB + BOND_EMB:]

    D = D_init
    for _ in range(PASSES + 1):
        msg = jnp.maximum(jnp.take(C, src, axis=0)
                          + jnp.take(D, edge_attr, axis=0), 0.0)
        agg = jax.ops.segment_sum(msg, dst, num_segments=N)
        C = baseN + agg @ Wa
        D = D_upd

    node_state = agg
    mol_state = jax.ops.segment_sum(node_state, batch, num_segments=NGRAPH)
    return _mlp_head(mol_state, W1, b1, W2, b2)


# speedup vs baseline: 4.7045x; 4.7045x over previous
import functools

import jax
import jax.numpy as jnp
from jax import lax
from jax.experimental import pallas as pl
from jax.experimental.pallas import tpu as pltpu
from jax.experimental.pallas import tpu_sc as plsc

N = 50000
E = 800000
ATOM_EMB = 64
BOND_EMB = 16
MSG = 64
PASSES = 3
HID = 64
NGRAPH = 1024

HALF = MSG // 2
N_PAD = 50048
ROWS_PER_TILE = N_PAD // 16
E_PAD = 802816
EROWS = E_PAD // 128
WIN = 128
WPM = 56
NMACRO = 7
TILE_EROWS = EROWS // 16
MOL_PAD = NGRAPH + 8
NBLK = 50
BLK = N // NBLK


def _embed_body(xf_ref, atom_ref, wia_ref, wua_ref, c0_ref, base_ref):
    x_col = xf_ref[...]
    iota = lax.broadcasted_iota(jnp.int32, (BLK, 128), 1).astype(jnp.float32)
    oh = (x_col == iota).astype(jnp.float32)
    atom = jnp.dot(oh, atom_ref[...], preferred_element_type=jnp.float32)
    c0_ref[...] = jnp.dot(atom, wia_ref[...],
                          preferred_element_type=jnp.float32)
    base_ref[...] = jnp.dot(atom, wua_ref[...],
                            preferred_element_type=jnp.float32)


def _embed(x_f, atom_pad, Wi_a, Wu_a):
    return pl.pallas_call(
        _embed_body,
        grid=(NBLK,),
        in_specs=[
            pl.BlockSpec((BLK, 1), lambda i: (i, 0)),
            pl.BlockSpec((128, MSG), lambda i: (0, 0)),
            pl.BlockSpec((MSG, MSG), lambda i: (0, 0)),
            pl.BlockSpec((MSG, MSG), lambda i: (0, 0)),
        ],
        out_specs=[
            pl.BlockSpec((BLK, MSG), lambda i: (i, 0)),
            pl.BlockSpec((BLK, MSG), lambda i: (i, 0)),
        ],
        out_shape=[
            jax.ShapeDtypeStruct((N, MSG), jnp.float32),
            jax.ShapeDtypeStruct((N, MSG), jnp.float32),
        ],
    )(x_f, atom_pad, Wi_a, Wu_a)


def _idx_body(src_ref, attr_ref, out_ref):
    g = attr_ref[...] * N + src_ref[...]
    out_ref[0] = g
    out_ref[1] = g + 4 * N


def _build_idx(src_p, attr_p):
    return pl.pallas_call(
        _idx_body,
        out_shape=jax.ShapeDtypeStruct((2, EROWS, 128), jnp.int32),
    )(src_p, attr_p)


def _d_row(bond_ref, wb_ref, b_ref, a):
    d_all = jnp.dot(bond_ref[...], wb_ref[...],
                    preferred_element_type=jnp.float32) + b_ref[...][None, :]
    rowids = lax.broadcasted_iota(jnp.int32, d_all.shape, 0)
    return jnp.sum(jnp.where(rowids == a, d_all, 0.0), axis=0, keepdims=True)


def _t0_body(c_ref, bond_ref, wb_ref, b_ref, out_ref):
    h = pl.program_id(1)
    a = pl.program_id(2)
    t = jnp.maximum(c_ref[...] + _d_row(bond_ref, wb_ref, b_ref, a), 0.0)
    out_ref[...] = jnp.where(h == 0, t[:, :HALF], t[:, HALF:])


def _build_t0(C0, bond_table, W_b, b):
    return pl.pallas_call(
        _t0_body,
        grid=(NBLK, 2, 4),
        in_specs=[
            pl.BlockSpec((BLK, MSG), lambda i, h, a: (i, 0)),
            pl.BlockSpec((4, BOND_EMB), lambda i, h, a: (0, 0)),
            pl.BlockSpec((BOND_EMB, MSG), lambda i, h, a: (0, 0)),
            pl.BlockSpec((MSG,), lambda i, h, a: (0,)),
        ],
        out_specs=pl.BlockSpec(
            (BLK, HALF), lambda i, h, a: (h * 4 * NBLK + a * NBLK + i, 0)),
        out_shape=jax.ShapeDtypeStruct((8 * N, HALF), jnp.float32),
    )(C0, bond_table, W_b, b)


def _tupd_body(base_ref, agg_lo_ref, agg_hi_ref, wa_lo_ref, wa_hi_ref,
               bond_ref, wb_ref, b_ref, out_ref):
    h = pl.program_id(1)
    a = pl.program_id(2)
    c = (base_ref[...]
         + jnp.dot(agg_lo_ref[0], wa_lo_ref[...],
                   preferred_element_type=jnp.float32)
         + jnp.dot(agg_hi_ref[0], wa_hi_ref[...],
                   preferred_element_type=jnp.float32))
    t = jnp.maximum(c + _d_row(bond_ref, wb_ref, b_ref, a), 0.0)
    out_ref[...] = jnp.where(h == 0, t[:, :HALF], t[:, HALF:])


def _build_tupd(baseN, agg2, Wa_lo, Wa_hi, bond_table, W_b, b):
    return pl.pallas_call(
        _tupd_body,
        grid=(NBLK, 2, 4),
        in_specs=[
            pl.BlockSpec((BLK, MSG), lambda i, h, a: (i, 0)),
            pl.BlockSpec((1, BLK, HALF), lambda i, h, a: (0, i, 0)),
            pl.BlockSpec((1, BLK, HALF), lambda i, h, a: (1, i, 0)),
            pl.BlockSpec((HALF, MSG), lambda i, h, a: (0, 0)),
            pl.BlockSpec((HALF, MSG), lambda i, h, a: (0, 0)),
            pl.BlockSpec((4, BOND_EMB), lambda i, h, a: (0, 0)),
            pl.BlockSpec((BOND_EMB, MSG), lambda i, h, a: (0, 0)),
            pl.BlockSpec((MSG,), lambda i, h, a: (0,)),
        ],
        out_specs=pl.BlockSpec(
            (BLK, HALF), lambda i, h, a: (h * 4 * NBLK + a * NBLK + i, 0)),
        out_shape=jax.ShapeDtypeStruct((8 * N, HALF), jnp.float32),
    )(baseN, agg2, agg2, Wa_lo, Wa_hi, bond_table, W_b, b)


def _mlp_body(mol_ref, w1_ref, b1_ref, w2_ref, b2_ref, out_ref):
    m = jnp.concatenate([mol_ref[0], mol_ref[1]], axis=1)
    h = jnp.maximum(
        jnp.dot(m, w1_ref[...], preferred_element_type=jnp.float32)
        + b1_ref[...][None, :], 0.0)
    out = jnp.dot(h, w2_ref[...], preferred_element_type=jnp.float32) \
        + b2_ref[...][None, :]
    out_ref[...] = out[:, 0]


def _mlp_head(mol2, W1, b1, W2, b2):
    return pl.pallas_call(
        _mlp_body,
        out_shape=jax.ShapeDtypeStruct((NGRAPH,), jnp.float32),
    )(mol2, W1, b1, W2, b2)


_SC_MESH = plsc.VectorSubcoreMesh(core_axis_name="c", subcore_axis_name="s")
_SC_PARAMS = pltpu.CompilerParams(use_tc_tiling_on_sc=False)


@functools.partial(
    pl.kernel,
    mesh=_SC_MESH,
    out_type=jax.ShapeDtypeStruct((2, N_PAD, HALF), jnp.float32),
    scratch_types=[
        pltpu.VMEM((WPM, 128), jnp.int32),
        pltpu.VMEM((WPM, 128), jnp.int32),
        pltpu.VMEM((WIN, HALF), jnp.float32),
        pltpu.VMEM_SHARED((N_PAD, HALF), jnp.float32),
        pltpu.SemaphoreType.DMA,
        pltpu.SemaphoreType.DMA,
    ],
    compiler_params=_SC_PARAMS,
)
def _sc_round(t8, idx2, dstr, zeros, out, gi, di, rows, shared, sem1, sem2):
    c = lax.axis_index("c")
    s = lax.axis_index("s")
    pltpu.async_copy(
        zeros, shared.at[pl.ds(s * ROWS_PER_TILE, ROWS_PER_TILE)], sem1
    ).wait()
    plsc.subcore_barrier()

    row0 = s * TILE_EROWS

    @pl.loop(0, NMACRO)
    def _macro(k):
        base = row0 + k * WPM
        pltpu.async_copy(idx2.at[c, pl.ds(base, WPM)], gi, sem1).wait()
        pltpu.async_copy(dstr.at[pl.ds(base, WPM)], di, sem1).wait()

        @pl.loop(0, WPM)
        def _win(j):
            pltpu.async_copy(t8.at[gi.at[j]], rows, sem2).wait()
            pltpu.sync_copy(rows, shared.at[di.at[j]], add=True)

    plsc.subcore_barrier()
    pltpu.async_copy(
        shared.at[pl.ds(s * ROWS_PER_TILE, ROWS_PER_TILE)],
        out.at[c].at[pl.ds(s * ROWS_PER_TILE, ROWS_PER_TILE)],
        sem1,
    ).wait()


_MOL_ROWS_A = 3200
_MOL_ROWS_B = 2048


@functools.partial(
    pl.kernel,
    mesh=_SC_MESH,
    out_type=jax.ShapeDtypeStruct((2, NGRAPH, HALF), jnp.float32),
    scratch_types=[
        pltpu.VMEM((WIN,), jnp.int32),
        pltpu.VMEM((WIN, HALF), jnp.float32),
        pltpu.VMEM_SHARED((MOL_PAD, HALF), jnp.float32),
        pltpu.SemaphoreType.DMA,
        pltpu.SemaphoreType.DMA,
    ],
    compiler_params=_SC_PARAMS,
)
def _sc_mol(agg2, batch_p, zeros, out, bi, rows, shared, sem1, sem2):
    c = lax.axis_index("c")
    s = lax.axis_index("s")
    pltpu.async_copy(
        zeros.at[pl.ds(0, NGRAPH // 16)],
        shared.at[pl.ds(s * (NGRAPH // 16), NGRAPH // 16)], sem1,
    ).wait()
    plsc.subcore_barrier()

    n_win = jnp.where(s < 15, _MOL_ROWS_A // WIN, _MOL_ROWS_B // WIN)

    @pl.loop(0, n_win)
    def _win(j):
        base = s * _MOL_ROWS_A + j * WIN
        pltpu.async_copy(batch_p.at[pl.ds(base, WIN)], bi, sem1).wait()
        pltpu.async_copy(agg2.at[c].at[pl.ds(base, WIN)], rows, sem2).wait()
        pltpu.sync_copy(rows, shared.at[bi], add=True)

    plsc.subcore_barrier()
    pltpu.async_copy(
        shared.at[pl.ds(s * (NGRAPH // 16), NGRAPH // 16)],
        out.at[c].at[pl.ds(s * (NGRAPH // 16), NGRAPH // 16)],
        sem1,
    ).wait()


def kernel(x, edge_index, edge_attr, batch, atom_table, bond_table,
           W_init, b_init, W_upd, b_upd, W1, b1, W2, b2):
    src = edge_index[0].astype(jnp.int32)
    dst = edge_index[1].astype(jnp.int32)
    attr = edge_attr.astype(jnp.int32)

    src_p = jnp.pad(src, (0, E_PAD - E)).reshape(EROWS, 128)
    attr_p = jnp.pad(attr, (0, E_PAD - E)).reshape(EROWS, 128)
    dst_p = jnp.pad(dst, (0, E_PAD - E),
                    constant_values=N_PAD - 8).reshape(EROWS, 128)
    x_f = x.astype(jnp.float32).reshape(N, 1)
    atom_pad = jnp.pad(atom_table, ((0, 128 - atom_table.shape[0]), (0, 0)))
    batch_p = jnp.pad(batch.astype(jnp.int32), (0, N_PAD - N),
                      constant_values=NGRAPH)
    zeros_hbm = jnp.zeros((ROWS_PER_TILE, HALF), jnp.float32)

    Wi_a = W_init[:ATOM_EMB]
    Wu_a = W_upd[:ATOM_EMB]
    Wi_b = W_init[ATOM_EMB:]
    Wu_b = W_upd[ATOM_EMB:ATOM_EMB + BOND_EMB]
    Wa = W_upd[ATOM_EMB + BOND_EMB:]
    Wa_lo = Wa[:HALF]
    Wa_hi = Wa[HALF:]

    idx2 = _build_idx(src_p, attr_p)
    C0, baseN = _embed(x_f, atom_pad, Wi_a, Wu_a)

    t8 = _build_t0(C0, bond_table, Wi_b, b_init)
    agg2 = _sc_round(t8, idx2, dst_p, zeros_hbm)
    for _ in range(PASSES):
        t8 = _build_tupd(baseN, agg2, Wa_lo, Wa_hi, bond_table, Wu_b, b_upd)
        agg2 = _sc_round(t8, idx2, dst_p, zeros_hbm)

    mol2 = _sc_mol(agg2, batch_p, zeros_hbm)
    return _mlp_head(mol2, W1, b1, W2, b2)


# speedup vs baseline: 5.1296x; 1.0904x over previous
import functools

import jax
import jax.numpy as jnp
from jax import lax
from jax.experimental import pallas as pl
from jax.experimental.pallas import tpu as pltpu
from jax.experimental.pallas import tpu_sc as plsc

N = 50000
E = 800000
ATOM_EMB = 64
BOND_EMB = 16
MSG = 64
PASSES = 3
HID = 64
NGRAPH = 1024

HALF = MSG // 2
N_PAD = 50048
ROWS_PER_TILE = N_PAD // 16
E_PAD = 802816
EROWS = E_PAD // 128
WIN = 128
WPM = 56
NMACRO = 7
TILE_EROWS = EROWS // 16
MOL_PAD = NGRAPH + 8
NBLK = 50
BLK = N // NBLK


def _embed_body(xf_ref, atom_ref, wia_ref, wua_ref, c0_ref, base_ref):
    x_col = xf_ref[...]
    iota = lax.broadcasted_iota(jnp.int32, (BLK, 128), 1).astype(jnp.float32)
    oh = (x_col == iota).astype(jnp.float32)
    atom = jnp.dot(oh, atom_ref[...], preferred_element_type=jnp.float32)
    c0_ref[...] = jnp.dot(atom, wia_ref[...],
                          preferred_element_type=jnp.float32)
    base_ref[...] = jnp.dot(atom, wua_ref[...],
                            preferred_element_type=jnp.float32)


def _embed(x_f, atom_pad, Wi_a, Wu_a):
    return pl.pallas_call(
        _embed_body,
        grid=(NBLK,),
        in_specs=[
            pl.BlockSpec((BLK, 1), lambda i: (i, 0)),
            pl.BlockSpec((128, MSG), lambda i: (0, 0)),
            pl.BlockSpec((MSG, MSG), lambda i: (0, 0)),
            pl.BlockSpec((MSG, MSG), lambda i: (0, 0)),
        ],
        out_specs=[
            pl.BlockSpec((BLK, MSG), lambda i: (i, 0)),
            pl.BlockSpec((BLK, MSG), lambda i: (i, 0)),
        ],
        out_shape=[
            jax.ShapeDtypeStruct((N, MSG), jnp.float32),
            jax.ShapeDtypeStruct((N, MSG), jnp.float32),
        ],
    )(x_f, atom_pad, Wi_a, Wu_a)


def _idx_body(src_ref, attr_ref, out_ref):
    g = attr_ref[...] * N + src_ref[...]
    out_ref[0] = g
    out_ref[1] = g + 4 * N


def _build_idx(src_p, attr_p):
    return pl.pallas_call(
        _idx_body,
        out_shape=jax.ShapeDtypeStruct((2, EROWS, 128), jnp.int32),
    )(src_p, attr_p)


def _d_row(bond_ref, wb_ref, b_ref, a):
    d_all = jnp.dot(bond_ref[...], wb_ref[...],
                    preferred_element_type=jnp.float32) + b_ref[...][None, :]
    rowids = lax.broadcasted_iota(jnp.int32, d_all.shape, 0)
    return jnp.sum(jnp.where(rowids == a, d_all, 0.0), axis=0, keepdims=True)


def _t0_body(c_ref, bond_ref, wb_ref, b_ref, out_ref):
    h = pl.program_id(1)
    a = pl.program_id(2)
    t = jnp.maximum(c_ref[...] + _d_row(bond_ref, wb_ref, b_ref, a), 0.0)
    out_ref[...] = jnp.where(h == 0, t[:, :HALF], t[:, HALF:])


def _build_t0(C0, bond_table, W_b, b):
    return pl.pallas_call(
        _t0_body,
        grid=(NBLK, 2, 4),
        in_specs=[
            pl.BlockSpec((BLK, MSG), lambda i, h, a: (i, 0)),
            pl.BlockSpec((4, BOND_EMB), lambda i, h, a: (0, 0)),
            pl.BlockSpec((BOND_EMB, MSG), lambda i, h, a: (0, 0)),
            pl.BlockSpec((MSG,), lambda i, h, a: (0,)),
        ],
        out_specs=pl.BlockSpec(
            (BLK, HALF), lambda i, h, a: (h * 4 * NBLK + a * NBLK + i, 0)),
        out_shape=jax.ShapeDtypeStruct((8 * N, HALF), jnp.float32),
    )(C0, bond_table, W_b, b)


def _tupd_body(base_ref, agg_lo_ref, agg_hi_ref, wa_lo_ref, wa_hi_ref,
               bond_ref, wb_ref, b_ref, out_ref):
    h = pl.program_id(1)
    a = pl.program_id(2)
    c = (base_ref[...]
         + jnp.dot(agg_lo_ref[0], wa_lo_ref[...],
                   preferred_element_type=jnp.float32)
         + jnp.dot(agg_hi_ref[0], wa_hi_ref[...],
                   preferred_element_type=jnp.float32))
    t = jnp.maximum(c + _d_row(bond_ref, wb_ref, b_ref, a), 0.0)
    out_ref[...] = jnp.where(h == 0, t[:, :HALF], t[:, HALF:])


def _build_tupd(baseN, agg2, Wa_lo, Wa_hi, bond_table, W_b, b):
    return pl.pallas_call(
        _tupd_body,
        grid=(NBLK, 2, 4),
        in_specs=[
            pl.BlockSpec((BLK, MSG), lambda i, h, a: (i, 0)),
            pl.BlockSpec((1, BLK, HALF), lambda i, h, a: (0, i, 0)),
            pl.BlockSpec((1, BLK, HALF), lambda i, h, a: (1, i, 0)),
            pl.BlockSpec((HALF, MSG), lambda i, h, a: (0, 0)),
            pl.BlockSpec((HALF, MSG), lambda i, h, a: (0, 0)),
            pl.BlockSpec((4, BOND_EMB), lambda i, h, a: (0, 0)),
            pl.BlockSpec((BOND_EMB, MSG), lambda i, h, a: (0, 0)),
            pl.BlockSpec((MSG,), lambda i, h, a: (0,)),
        ],
        out_specs=pl.BlockSpec(
            (BLK, HALF), lambda i, h, a: (h * 4 * NBLK + a * NBLK + i, 0)),
        out_shape=jax.ShapeDtypeStruct((8 * N, HALF), jnp.float32),
    )(baseN, agg2, agg2, Wa_lo, Wa_hi, bond_table, W_b, b)


def _mlp_body(mol_ref, w1_ref, b1_ref, w2_ref, b2_ref, out_ref):
    m = jnp.concatenate([mol_ref[0], mol_ref[1]], axis=1)
    h = jnp.maximum(
        jnp.dot(m, w1_ref[...], preferred_element_type=jnp.float32)
        + b1_ref[...][None, :], 0.0)
    out = jnp.dot(h, w2_ref[...], preferred_element_type=jnp.float32) \
        + b2_ref[...][None, :]
    out_ref[...] = out[:, 0]


def _mlp_head(mol2, W1, b1, W2, b2):
    return pl.pallas_call(
        _mlp_body,
        out_shape=jax.ShapeDtypeStruct((NGRAPH,), jnp.float32),
    )(mol2, W1, b1, W2, b2)


_SC_MESH = plsc.VectorSubcoreMesh(core_axis_name="c", subcore_axis_name="s")
_SC_PARAMS = pltpu.CompilerParams(use_tc_tiling_on_sc=False)


@functools.partial(
    pl.kernel,
    mesh=_SC_MESH,
    out_type=jax.ShapeDtypeStruct((2, N_PAD, HALF), jnp.float32),
    scratch_types=[
        pltpu.VMEM((WPM, 128), jnp.int32),
        pltpu.VMEM((WPM, 128), jnp.int32),
        pltpu.VMEM((WIN, HALF), jnp.float32),
        pltpu.VMEM((WIN, HALF), jnp.float32),
        pltpu.VMEM_SHARED((N_PAD, HALF), jnp.float32),
        pltpu.SemaphoreType.DMA,
        pltpu.SemaphoreType.DMA,
    ],
    compiler_params=_SC_PARAMS,
)
def _sc_round(t8, idx2, dstr, zeros, out, gi, di, rows0, rows1, shared,
              sem1, sem2):
    c = lax.axis_index("c")
    s = lax.axis_index("s")
    pltpu.async_copy(
        zeros, shared.at[pl.ds(s * ROWS_PER_TILE, ROWS_PER_TILE)], sem1
    ).wait()
    plsc.subcore_barrier()

    row0 = s * TILE_EROWS

    def _gather(j, buf):
        return pltpu.make_async_copy(t8.at[gi.at[j]], buf, sem2)

    @pl.loop(0, NMACRO)
    def _macro(k):
        base = row0 + k * WPM
        pltpu.async_copy(idx2.at[c, pl.ds(base, WPM)], gi, sem1).wait()
        pltpu.async_copy(dstr.at[pl.ds(base, WPM)], di, sem1).wait()
        _gather(0, rows0).start()

        @pl.loop(0, WPM, step=2)
        def _pair(j):
            _gather(j, rows0).wait()
            _gather(j + 1, rows1).start()
            pltpu.sync_copy(rows0, shared.at[di.at[j]], add=True)
            _gather(j + 1, rows1).wait()

            @pl.when(j + 2 < WPM)
            def _():
                _gather(j + 2, rows0).start()

            pltpu.sync_copy(rows1, shared.at[di.at[j + 1]], add=True)

    plsc.subcore_barrier()
    pltpu.async_copy(
        shared.at[pl.ds(s * ROWS_PER_TILE, ROWS_PER_TILE)],
        out.at[c].at[pl.ds(s * ROWS_PER_TILE, ROWS_PER_TILE)],
        sem1,
    ).wait()


_MOL_ROWS_A = 3200
_MOL_ROWS_B = 2048


@functools.partial(
    pl.kernel,
    mesh=_SC_MESH,
    out_type=jax.ShapeDtypeStruct((2, NGRAPH, HALF), jnp.float32),
    scratch_types=[
        pltpu.VMEM((WIN,), jnp.int32),
        pltpu.VMEM((WIN, HALF), jnp.float32),
        pltpu.VMEM_SHARED((MOL_PAD, HALF), jnp.float32),
        pltpu.SemaphoreType.DMA,
        pltpu.SemaphoreType.DMA,
    ],
    compiler_params=_SC_PARAMS,
)
def _sc_mol(agg2, batch_p, zeros, out, bi, rows, shared, sem1, sem2):
    c = lax.axis_index("c")
    s = lax.axis_index("s")
    pltpu.async_copy(
        zeros.at[pl.ds(0, NGRAPH // 16)],
        shared.at[pl.ds(s * (NGRAPH // 16), NGRAPH // 16)], sem1,
    ).wait()
    plsc.subcore_barrier()

    n_win = jnp.where(s < 15, _MOL_ROWS_A // WIN, _MOL_ROWS_B // WIN)

    @pl.loop(0, n_win)
    def _win(j):
        base = s * _MOL_ROWS_A + j * WIN
        pltpu.async_copy(batch_p.at[pl.ds(base, WIN)], bi, sem1).wait()
        pltpu.async_copy(agg2.at[c].at[pl.ds(base, WIN)], rows, sem2).wait()
        pltpu.sync_copy(rows, shared.at[bi], add=True)

    plsc.subcore_barrier()
    pltpu.async_copy(
        shared.at[pl.ds(s * (NGRAPH // 16), NGRAPH // 16)],
        out.at[c].at[pl.ds(s * (NGRAPH // 16), NGRAPH // 16)],
        sem1,
    ).wait()


def kernel(x, edge_index, edge_attr, batch, atom_table, bond_table,
           W_init, b_init, W_upd, b_upd, W1, b1, W2, b2):
    src = edge_index[0].astype(jnp.int32)
    dst = edge_index[1].astype(jnp.int32)
    attr = edge_attr.astype(jnp.int32)

    src_p = jnp.pad(src, (0, E_PAD - E)).reshape(EROWS, 128)
    attr_p = jnp.pad(attr, (0, E_PAD - E)).reshape(EROWS, 128)
    dst_p = jnp.pad(dst, (0, E_PAD - E),
                    constant_values=N_PAD - 8).reshape(EROWS, 128)
    x_f = x.astype(jnp.float32).reshape(N, 1)
    atom_pad = jnp.pad(atom_table, ((0, 128 - atom_table.shape[0]), (0, 0)))
    batch_p = jnp.pad(batch.astype(jnp.int32), (0, N_PAD - N),
                      constant_values=NGRAPH)
    zeros_hbm = jnp.zeros((ROWS_PER_TILE, HALF), jnp.float32)

    Wi_a = W_init[:ATOM_EMB]
    Wu_a = W_upd[:ATOM_EMB]
    Wi_b = W_init[ATOM_EMB:]
    Wu_b = W_upd[ATOM_EMB:ATOM_EMB + BOND_EMB]
    Wa = W_upd[ATOM_EMB + BOND_EMB:]
    Wa_lo = Wa[:HALF]
    Wa_hi = Wa[HALF:]

    idx2 = _build_idx(src_p, attr_p)
    C0, baseN = _embed(x_f, atom_pad, Wi_a, Wu_a)

    t8 = _build_t0(C0, bond_table, Wi_b, b_init)
    agg2 = _sc_round(t8, idx2, dst_p, zeros_hbm)
    for _ in range(PASSES):
        t8 = _build_tupd(baseN, agg2, Wa_lo, Wa_hi, bond_table, Wu_b, b_upd)
        agg2 = _sc_round(t8, idx2, dst_p, zeros_hbm)

    mol2 = _sc_mol(agg2, batch_p, zeros_hbm)
    return _mlp_head(mol2, W1, b1, W2, b2)


# speedup vs baseline: 7.9911x; 1.5578x over previous
import functools

import jax
import jax.numpy as jnp
from jax import lax
from jax.experimental import pallas as pl
from jax.experimental.pallas import tpu as pltpu
from jax.experimental.pallas import tpu_sc as plsc

N = 50000
E = 800000
ATOM_EMB = 64
BOND_EMB = 16
MSG = 64
PASSES = 3
HID = 64
NGRAPH = 1024

HALF = MSG // 2
N_T = 51200
ROWS_PER_TILE = N_T // 16
E_PAD = 802816
EROWS = E_PAD // 128
WIN = 128
WPM = 56
NMACRO = 7
TILE_EROWS = EROWS // 16
MOL_PAD = NGRAPH + 8
QTR = N_T // 4
TBLK4 = 2560
TNBLK = QTR // TBLK4
EBLK = 1024
ENBLK = N_T // EBLK
DST_JUNK = N_T - 8


def _embed_body(xf_ref, atom_ref, wia_ref, wua_ref, c0_ref, base_ref):
    x_col = xf_ref[...]
    iota = lax.broadcasted_iota(jnp.int32, (EBLK, 128), 1).astype(jnp.float32)
    oh = (x_col == iota).astype(jnp.float32)
    atom = jnp.dot(oh, atom_ref[...], preferred_element_type=jnp.float32)
    c0_ref[...] = jnp.dot(atom, wia_ref[...],
                          preferred_element_type=jnp.float32)
    base_ref[...] = jnp.dot(atom, wua_ref[...],
                            preferred_element_type=jnp.float32)


def _embed(x_f, atom_pad, Wi_a, Wu_a):
    return pl.pallas_call(
        _embed_body,
        grid=(ENBLK,),
        in_specs=[
            pl.BlockSpec((EBLK, 1), lambda i: (i, 0)),
            pl.BlockSpec((128, MSG), lambda i: (0, 0)),
            pl.BlockSpec((MSG, MSG), lambda i: (0, 0)),
            pl.BlockSpec((MSG, MSG), lambda i: (0, 0)),
        ],
        out_specs=[
            pl.BlockSpec((EBLK, MSG), lambda i: (i, 0)),
            pl.BlockSpec((EBLK, MSG), lambda i: (i, 0)),
        ],
        out_shape=[
            jax.ShapeDtypeStruct((N_T, MSG), jnp.float32),
            jax.ShapeDtypeStruct((N_T, MSG), jnp.float32),
        ],
    )(x_f, atom_pad, Wi_a, Wu_a)


def _idx_body(src_ref, attr_ref, out_ref):
    src = src_ref[...]
    q = src // QTR
    m = src - q * QTR
    g = attr_ref[...] * N_T + 4 * m + q
    out_ref[0] = g
    out_ref[1] = g + 4 * N_T


def _build_idx(src_p, attr_p):
    return pl.pallas_call(
        _idx_body,
        out_shape=jax.ShapeDtypeStruct((2, EROWS, 128), jnp.int32),
    )(src_p, attr_p)


def _d_row(bond_ref, wb_ref, b_ref, a):
    d_all = jnp.dot(bond_ref[...], wb_ref[...],
                    preferred_element_type=jnp.float32) + b_ref[...][None, :]
    rowids = lax.broadcasted_iota(jnp.int32, d_all.shape, 0)
    return jnp.sum(jnp.where(rowids == a, d_all, 0.0), axis=0, keepdims=True)


def _t0_body(c0_ref, c1_ref, c2_ref, c3_ref, bond_ref, wb_ref, b_ref,
             out_ref):
    h = pl.program_id(1)
    a = pl.program_id(2)
    d = _d_row(bond_ref, wb_ref, b_ref, a)
    parts = []
    for cq_ref in (c0_ref, c1_ref, c2_ref, c3_ref):
        t = jnp.maximum(cq_ref[...] + d, 0.0)
        parts.append(jnp.where(h == 0, t[:, :HALF], t[:, HALF:]))
    out_ref[...] = jnp.concatenate(parts, axis=1)


def _quarter_spec(q):
    return pl.BlockSpec((TBLK4, MSG), lambda i, h, a, q=q: (q * TNBLK + i, 0))


def _quarter_spec3(q):
    return pl.BlockSpec((2, TBLK4, HALF),
                        lambda i, h, a, q=q: (0, q * TNBLK + i, 0))


def _build_t0(C0, bond_table, W_b, b):
    return pl.pallas_call(
        _t0_body,
        grid=(TNBLK, 2, 4),
        in_specs=[
            _quarter_spec(0), _quarter_spec(1),
            _quarter_spec(2), _quarter_spec(3),
            pl.BlockSpec((4, BOND_EMB), lambda i, h, a: (0, 0)),
            pl.BlockSpec((BOND_EMB, MSG), lambda i, h, a: (0, 0)),
            pl.BlockSpec((MSG,), lambda i, h, a: (0,)),
        ],
        out_specs=pl.BlockSpec(
            (TBLK4, 128),
            lambda i, h, a: (h * 4 * TNBLK + a * TNBLK + i, 0)),
        out_shape=jax.ShapeDtypeStruct((2 * N_T, 128), jnp.float32),
    )(C0, C0, C0, C0, bond_table, W_b, b)


def _tupd_body(b0_ref, b1_ref, b2_ref, b3_ref, a0_ref, a1_ref, a2_ref,
               a3_ref, wa_lo_ref, wa_hi_ref, bond_ref, wb_ref, b_ref,
               out_ref):
    h = pl.program_id(1)
    a = pl.program_id(2)
    d = _d_row(bond_ref, wb_ref, b_ref, a)
    parts = []
    for bq_ref, aq_ref in ((b0_ref, a0_ref), (b1_ref, a1_ref),
                           (b2_ref, a2_ref), (b3_ref, a3_ref)):
        c = (bq_ref[...]
             + jnp.dot(aq_ref[0], wa_lo_ref[...],
                       preferred_element_type=jnp.float32)
             + jnp.dot(aq_ref[1], wa_hi_ref[...],
                       preferred_element_type=jnp.float32))
        t = jnp.maximum(c + d, 0.0)
        parts.append(jnp.where(h == 0, t[:, :HALF], t[:, HALF:]))
    out_ref[...] = jnp.concatenate(parts, axis=1)


def _build_tupd(baseN, agg2, Wa_lo, Wa_hi, bond_table, W_b, b):
    return pl.pallas_call(
        _tupd_body,
        grid=(TNBLK, 2, 4),
        in_specs=[
            _quarter_spec(0), _quarter_spec(1),
            _quarter_spec(2), _quarter_spec(3),
            _quarter_spec3(0), _quarter_spec3(1),
            _quarter_spec3(2), _quarter_spec3(3),
            pl.BlockSpec((HALF, MSG), lambda i, h, a: (0, 0)),
            pl.BlockSpec((HALF, MSG), lambda i, h, a: (0, 0)),
            pl.BlockSpec((4, BOND_EMB), lambda i, h, a: (0, 0)),
            pl.BlockSpec((BOND_EMB, MSG), lambda i, h, a: (0, 0)),
            pl.BlockSpec((MSG,), lambda i, h, a: (0,)),
        ],
        out_specs=pl.BlockSpec(
            (TBLK4, 128),
            lambda i, h, a: (h * 4 * TNBLK + a * TNBLK + i, 0)),
        out_shape=jax.ShapeDtypeStruct((2 * N_T, 128), jnp.float32),
    )(baseN, baseN, baseN, baseN, agg2, agg2, agg2, agg2,
      Wa_lo, Wa_hi, bond_table, W_b, b)


def _mlp_body(mol_ref, w1_ref, b1_ref, w2_ref, b2_ref, out_ref):
    m = jnp.concatenate([mol_ref[0], mol_ref[1]], axis=1)
    h = jnp.maximum(
        jnp.dot(m, w1_ref[...], preferred_element_type=jnp.float32)
        + b1_ref[...][None, :], 0.0)
    out = jnp.dot(h, w2_ref[...], preferred_element_type=jnp.float32) \
        + b2_ref[...][None, :]
    out_ref[...] = out[:, 0]


def _mlp_head(mol2, W1, b1, W2, b2):
    return pl.pallas_call(
        _mlp_body,
        out_shape=jax.ShapeDtypeStruct((NGRAPH,), jnp.float32),
    )(mol2, W1, b1, W2, b2)


_SC_MESH = plsc.VectorSubcoreMesh(core_axis_name="c", subcore_axis_name="s")
_SC_PARAMS = pltpu.CompilerParams(use_tc_tiling_on_sc=False)


@functools.partial(
    pl.kernel,
    mesh=_SC_MESH,
    out_type=jax.ShapeDtypeStruct((2, N_T, HALF), jnp.float32),
    scratch_types=[
        pltpu.VMEM((WPM, 128), jnp.int32),
        pltpu.VMEM((WPM, 128), jnp.int32),
        pltpu.VMEM((WIN, HALF), jnp.float32),
        pltpu.VMEM((WIN, HALF), jnp.float32),
        pltpu.VMEM_SHARED((N_T, HALF), jnp.float32),
        pltpu.SemaphoreType.DMA,
        pltpu.SemaphoreType.DMA,
    ],
    compiler_params=_SC_PARAMS,
)
def _sc_round(t8, idx2, dstr, zeros, out, gi, di, rows0, rows1, shared,
              sem1, sem2):
    c = lax.axis_index("c")
    s = lax.axis_index("s")
    pltpu.async_copy(
        zeros, shared.at[pl.ds(s * ROWS_PER_TILE, ROWS_PER_TILE)], sem1
    ).wait()
    plsc.subcore_barrier()

    row0 = s * TILE_EROWS

    def _gather(j, buf):
        return pltpu.make_async_copy(t8.at[gi.at[j]], buf, sem2)

    @pl.loop(0, NMACRO)
    def _macro(k):
        base = row0 + k * WPM
        pltpu.async_copy(idx2.at[c, pl.ds(base, WPM)], gi, sem1).wait()
        pltpu.async_copy(dstr.at[pl.ds(base, WPM)], di, sem1).wait()
        _gather(0, rows0).start()

        @pl.loop(0, WPM, step=2)
        def _pair(j):
            _gather(j, rows0).wait()
            _gather(j + 1, rows1).start()
            pltpu.sync_copy(rows0, shared.at[di.at[j]], add=True)
            _gather(j + 1, rows1).wait()

            @pl.when(j + 2 < WPM)
            def _():
                _gather(j + 2, rows0).start()

            pltpu.sync_copy(rows1, shared.at[di.at[j + 1]], add=True)

    plsc.subcore_barrier()
    pltpu.async_copy(
        shared.at[pl.ds(s * ROWS_PER_TILE, ROWS_PER_TILE)],
        out.at[c].at[pl.ds(s * ROWS_PER_TILE, ROWS_PER_TILE)],
        sem1,
    ).wait()


@functools.partial(
    pl.kernel,
    mesh=_SC_MESH,
    out_type=jax.ShapeDtypeStruct((2, NGRAPH, HALF), jnp.float32),
    scratch_types=[
        pltpu.VMEM((WIN,), jnp.int32),
        pltpu.VMEM((WIN, HALF), jnp.float32),
        pltpu.VMEM_SHARED((MOL_PAD, HALF), jnp.float32),
        pltpu.SemaphoreType.DMA,
        pltpu.SemaphoreType.DMA,
    ],
    compiler_params=_SC_PARAMS,
)
def _sc_mol(agg2, batch_p, zeros, out, bi, rows, shared, sem1, sem2):
    c = lax.axis_index("c")
    s = lax.axis_index("s")
    pltpu.async_copy(
        zeros.at[pl.ds(0, NGRAPH // 16)],
        shared.at[pl.ds(s * (NGRAPH // 16), NGRAPH // 16)], sem1,
    ).wait()
    plsc.subcore_barrier()

    @pl.loop(0, ROWS_PER_TILE // WIN)
    def _win(j):
        base = s * ROWS_PER_TILE + j * WIN
        pltpu.async_copy(batch_p.at[pl.ds(base, WIN)], bi, sem1).wait()
        pltpu.async_copy(agg2.at[c].at[pl.ds(base, WIN)], rows, sem2).wait()
        pltpu.sync_copy(rows, shared.at[bi], add=True)

    plsc.subcore_barrier()
    pltpu.async_copy(
        shared.at[pl.ds(s * (NGRAPH // 16), NGRAPH // 16)],
        out.at[c].at[pl.ds(s * (NGRAPH // 16), NGRAPH // 16)],
        sem1,
    ).wait()


def kernel(x, edge_index, edge_attr, batch, atom_table, bond_table,
           W_init, b_init, W_upd, b_upd, W1, b1, W2, b2):
    src = edge_index[0].astype(jnp.int32)
    dst = edge_index[1].astype(jnp.int32)
    attr = edge_attr.astype(jnp.int32)

    src_p = jnp.pad(src, (0, E_PAD - E)).reshape(EROWS, 128)
    attr_p = jnp.pad(attr, (0, E_PAD - E)).reshape(EROWS, 128)
    dst_p = jnp.pad(dst, (0, E_PAD - E),
                    constant_values=DST_JUNK).reshape(EROWS, 128)
    x_f = jnp.pad(x.astype(jnp.float32), (0, N_T - N)).reshape(N_T, 1)
    atom_pad = jnp.pad(atom_table, ((0, 128 - atom_table.shape[0]), (0, 0)))
    batch_p = jnp.pad(batch.astype(jnp.int32), (0, N_T - N),
                      constant_values=NGRAPH)
    zeros_hbm = jnp.zeros((ROWS_PER_TILE, HALF), jnp.float32)

    Wi_a = W_init[:ATOM_EMB]
    Wu_a = W_upd[:ATOM_EMB]
    Wi_b = W_init[ATOM_EMB:]
    Wu_b = W_upd[ATOM_EMB:ATOM_EMB + BOND_EMB]
    Wa = W_upd[ATOM_EMB + BOND_EMB:]
    Wa_lo = Wa[:HALF]
    Wa_hi = Wa[HALF:]

    idx2 = _build_idx(src_p, attr_p)
    C0, baseN = _embed(x_f, atom_pad, Wi_a, Wu_a)

    t8 = jnp.reshape(_build_t0(C0, bond_table, Wi_b, b_init),
                     (8 * N_T, HALF))
    agg2 = _sc_round(t8, idx2, dst_p, zeros_hbm)
    for _ in range(PASSES):
        t8 = jnp.reshape(
            _build_tupd(baseN, agg2, Wa_lo, Wa_hi, bond_table, Wu_b, b_upd),
            (8 * N_T, HALF))
        agg2 = _sc_round(t8, idx2, dst_p, zeros_hbm)

    mol2 = _sc_mol(agg2, batch_p, zeros_hbm)
    return _mlp_head(mol2, W1, b1, W2, b2)
